# baseline (speedup 1.0000x reference)
import functools

import jax
import jax.numpy as jnp
from jax import lax
from jax.experimental import pallas as pl
from jax.experimental.pallas import tpu as pltpu
from jax.experimental.pallas import tpu_sc as plsc


def _make_gather(V, D, B):
    info = plsc.get_sparse_core_info()
    NC, NS = info.num_cores, info.num_subcores
    NW = NC * NS
    assert B % (8 * NW) == 0
    b_per_w = B // NW
    mesh = plsc.VectorSubcoreMesh(core_axis_name="c", subcore_axis_name="s")

    @functools.partial(
        pl.kernel,
        mesh=mesh,
        out_type=jax.ShapeDtypeStruct((B, D), jnp.float32),
        scratch_types=[
            pltpu.VMEM((b_per_w,), jnp.int32),
            pltpu.VMEM((b_per_w, D), jnp.float32),
            pltpu.SemaphoreType.DMA,
        ],
        compiler_params=pltpu.CompilerParams(use_tc_tiling_on_sc=False),
    )
    def gather_k(table_hbm, idx_hbm, out_hbm, idx_v, rows_v, sem):
        wid = lax.axis_index("s") * NC + lax.axis_index("c")
        base = wid * b_per_w
        pltpu.sync_copy(idx_hbm.at[pl.ds(base, b_per_w)], idx_v)
        pltpu.async_copy(table_hbm.at[idx_v], rows_v, sem).wait()
        pltpu.sync_copy(rows_v, out_hbm.at[pl.ds(base, b_per_w)])

    return gather_k


def kernel(inputs, table):
    B = inputs.shape[0]
    V, D = table.shape
    gather_k = _make_gather(V, D, B)
    out = gather_k(table, inputs.astype(jnp.int32))
    return out.reshape(1, B * D)


# speedup vs baseline: 4.2338x; 4.2338x over previous
import functools

import jax
import jax.numpy as jnp
from jax import lax
from jax.experimental import pallas as pl
from jax.experimental.pallas import tpu as pltpu
from jax.experimental.pallas import tpu_sc as plsc

_LANES = 16
_RING = 8


def _make_gather(V, D, B):
    info = plsc.get_sparse_core_info()
    NC, NS = info.num_cores, info.num_subcores
    NW = NC * NS
    assert B % (8 * NW) == 0 and D == 32
    b_per_w = B // NW
    mesh = plsc.VectorSubcoreMesh(core_axis_name="c", subcore_axis_name="s")

    @functools.partial(
        pl.kernel,
        mesh=mesh,
        out_type=jax.ShapeDtypeStruct((B * D,), jnp.float32),
        scratch_types=[
            pltpu.VMEM((b_per_w,), jnp.int32),
            pltpu.VMEM((_RING, D, 128), jnp.float32),
            pltpu.VMEM((b_per_w * D,), jnp.float32),
            pltpu.SemaphoreType.DMA,
        ],
        compiler_params=pltpu.CompilerParams(needs_layout_passes=False),
    )
    def gather_k(t_hbm, idx_hbm, out_hbm, idx_v, ring, out_flat, sem):
        wid = lax.axis_index("s") * NC + lax.axis_index("c")
        base = wid * b_per_w
        pltpu.sync_copy(idx_hbm.at[pl.ds(base, b_per_w)], idx_v)

        lanes0 = lax.iota(jnp.int32, _LANES)
        lanes1 = lanes0 + _LANES
        c127 = jnp.full((_LANES,), 127, jnp.int32)

        strip_vecs = []
        col_vecs = []
        for g in range(b_per_w // _LANES):
            vec = idx_v[pl.ds(g * _LANES, _LANES)]
            strip_vecs.append(lax.shift_right_logical(vec, 7) * 128)
            col_vecs.append(lax.bitwise_and(vec, c127))

        def fetch(i):
            c0 = pl.multiple_of(strip_vecs[i // _LANES][i % _LANES], 128)
            return pltpu.async_copy(
                t_hbm.at[:, pl.ds(c0, 128)], ring.at[i % _RING], sem
            )

        pending = [fetch(i) for i in range(_RING)]
        for i in range(b_per_w):
            pending[i % _RING].wait()
            col = col_vecs[i // _LANES][i % _LANES]
            cv = jnp.full((_LANES,), col, jnp.int32)
            sv = jnp.full((_LANES,), i % _RING, jnp.int32)
            out_flat[pl.ds(i * D, _LANES)] = plsc.load_gather(
                ring, [sv, lanes0, cv]
            )
            out_flat[pl.ds(i * D + _LANES, _LANES)] = plsc.load_gather(
                ring, [sv, lanes1, cv]
            )
            if i + _RING < b_per_w:
                pending[i % _RING] = fetch(i + _RING)

        pltpu.sync_copy(out_flat, out_hbm.at[pl.ds(base * D, b_per_w * D)])

    return gather_k


def kernel(inputs, table):
    B = inputs.shape[0]
    V, D = table.shape
    gather_k = _make_gather(V, D, B)
    out = gather_k(table.T, inputs.astype(jnp.int32))
    return out.reshape(1, B * D)
